# baseline (speedup 1.0000x reference)
import functools

import jax
import jax.numpy as jnp
from jax import lax
from jax.experimental import pallas as pl
from jax.experimental.pallas import tpu as pltpu
from jax.experimental.pallas import tpu_sc as plsc

NC = 2
NS = 16
NW = NC * NS
CHUNK = 128


def _edge_agg(table, gidx2d, didx2d, zeros_pad, n_pad, d, n_chunks):
    rpt = n_pad // NS
    mesh = plsc.VectorSubcoreMesh(core_axis_name="c", subcore_axis_name="s")

    @functools.partial(
        pl.kernel,
        out_type=jax.ShapeDtypeStruct((NC, n_pad, d), jnp.float32),
        mesh=mesh,
        scratch_types=[
            pltpu.VMEM((n_chunks, CHUNK), jnp.int32),
            pltpu.VMEM((2, 8, CHUNK), jnp.int32),
            pltpu.VMEM((CHUNK, d), jnp.float32),
            pltpu.VMEM((CHUNK, d), jnp.float32),
            pltpu.VMEM_SHARED((n_pad, d), jnp.float32),
            pltpu.SemaphoreType.DMA,
            pltpu.SemaphoreType.DMA,
            pltpu.SemaphoreType.DMA,
            pltpu.SemaphoreType.DMA,
            pltpu.SemaphoreType.DMA,
        ],
    )
    def agg(table_hbm, gidx_hbm, didx_hbm, zeros_hbm, out_hbm,
            gidx_v, dring, rows_a, rows_b, acc,
            sem_ga, sem_gb, sem_sa, sem_sb, sem_i):
        c = lax.axis_index("c")
        s = lax.axis_index("s")
        wid = s * NC + c
        pltpu.sync_copy(zeros_hbm.at[pl.ds(s * rpt, rpt)],
                        acc.at[pl.ds(s * rpt, rpt)])
        pltpu.sync_copy(gidx_hbm.at[pl.ds(wid * n_chunks, n_chunks)], gidx_v)
        pltpu.sync_copy(didx_hbm.at[pl.ds(wid * n_chunks, 8)], dring.at[0])
        plsc.subcore_barrier()

        pltpu.async_copy(table_hbm.at[gidx_v.at[0]], rows_a, sem_ga)
        pltpu.async_copy(table_hbm.at[gidx_v.at[1]], rows_b, sem_gb)

        def step(t, carry):
            j = 2 * t
            grp = t >> 2
            q = grp & 1

            @pl.when((t & 3) == 0)
            def _():
                @pl.when(t > 0)
                def _():
                    pltpu.make_async_copy(
                        didx_hbm.at[pl.ds(0, 8)], dring.at[0], sem_i).wait()

                @pl.when(j + 8 < n_chunks)
                def _():
                    off = pl.multiple_of(wid * n_chunks + j + 8, 8)
                    pltpu.async_copy(didx_hbm.at[pl.ds(off, 8)],
                                     dring.at[1 - q], sem_i)

            pltpu.make_async_copy(table_hbm.at[gidx_v.at[0]], rows_a,
                                  sem_ga).wait()
            pltpu.async_copy(rows_a, acc.at[dring.at[q, 2 * (t & 3)]],
                             sem_sa, add=True)
            pltpu.make_async_copy(table_hbm.at[gidx_v.at[0]], rows_b,
                                  sem_gb).wait()
            pltpu.async_copy(rows_b, acc.at[dring.at[q, 2 * (t & 3) + 1]],
                             sem_sb, add=True)

            pltpu.make_async_copy(rows_a, acc.at[dring.at[0, 0]],
                                  sem_sa).wait()

            @pl.when(j + 2 < n_chunks)
            def _():
                pltpu.async_copy(table_hbm.at[gidx_v.at[j + 2]], rows_a,
                                 sem_ga)

            pltpu.make_async_copy(rows_b, acc.at[dring.at[0, 0]],
                                  sem_sb).wait()

            @pl.when(j + 3 < n_chunks)
            def _():
                pltpu.async_copy(table_hbm.at[gidx_v.at[j + 3]], rows_b,
                                 sem_gb)

            return carry

        lax.fori_loop(0, n_chunks // 2, step, 0)
        plsc.subcore_barrier()
        pltpu.sync_copy(acc.at[pl.ds(s * rpt, rpt)],
                        out_hbm.at[c, pl.ds(s * rpt, rpt)])

    return agg(table, gidx2d, didx2d, zeros_pad)


def _gather_rows(table, idx, d, per_w):
    g = idx.shape[0]
    mesh = plsc.VectorSubcoreMesh(core_axis_name="c", subcore_axis_name="s")

    @functools.partial(
        pl.kernel,
        out_type=jax.ShapeDtypeStruct((g, d), jnp.float32),
        mesh=mesh,
        scratch_types=[
            pltpu.VMEM((per_w,), jnp.int32),
            pltpu.VMEM((per_w, d), jnp.float32),
            pltpu.SemaphoreType.DMA,
        ],
    )
    def gat(table_hbm, idx_hbm, out_hbm, idx_v, rows_v, sem):
        wid = lax.axis_index("s") * NC + lax.axis_index("c")
        base = wid * per_w
        pltpu.sync_copy(idx_hbm.at[pl.ds(base, per_w)], idx_v)
        pltpu.async_copy(table_hbm.at[idx_v], rows_v, sem).wait()
        pltpu.sync_copy(rows_v, out_hbm.at[pl.ds(base, per_w)])

    return gat(table, idx)


def _mm(x, w, bn):
    n, k = x.shape
    m = w.shape[1]

    def body(x_ref, w_ref, o_ref):
        o_ref[...] = jnp.dot(x_ref[...], w_ref[...],
                             preferred_element_type=jnp.float32)

    return pl.pallas_call(
        body,
        grid=(n // bn,),
        in_specs=[pl.BlockSpec((bn, k), lambda i: (i, 0)),
                  pl.BlockSpec((k, m), lambda i: (0, 0))],
        out_specs=pl.BlockSpec((bn, m), lambda i: (i, 0)),
        out_shape=jax.ShapeDtypeStruct((n, m), jnp.float32),
    )(x, w)


def _mm_bias(x, w, b, bn):
    n, k = x.shape
    m = w.shape[1]

    def body(x_ref, w_ref, b_ref, o_ref):
        o_ref[...] = jnp.dot(x_ref[...], w_ref[...],
                             preferred_element_type=jnp.float32) + b_ref[...]

    return pl.pallas_call(
        body,
        grid=(n // bn,),
        in_specs=[pl.BlockSpec((bn, k), lambda i: (i, 0)),
                  pl.BlockSpec((k, m), lambda i: (0, 0)),
                  pl.BlockSpec((1, m), lambda i: (0, 0))],
        out_specs=pl.BlockSpec((bn, m), lambda i: (i, 0)),
        out_shape=jax.ShapeDtypeStruct((n, m), jnp.float32),
    )(x, w, b)


def _layer1b(parts, sl, wcat2, bn):
    n, d = sl.shape
    m = wcat2.shape[1]

    def body(p_ref, sl_ref, w_ref, o_ref):
        h = jnp.maximum(p_ref[0] + p_ref[1] + sl_ref[...], 0.0)
        o_ref[...] = jnp.dot(h, w_ref[...],
                             preferred_element_type=jnp.float32)

    return pl.pallas_call(
        body,
        grid=(n // bn,),
        in_specs=[pl.BlockSpec((2, bn, d), lambda i: (0, i, 0)),
                  pl.BlockSpec((bn, d), lambda i: (i, 0)),
                  pl.BlockSpec((d, m), lambda i: (0, 0))],
        out_specs=pl.BlockSpec((bn, m), lambda i: (i, 0)),
        out_shape=jax.ShapeDtypeStruct((n, m), jnp.float32),
    )(parts, sl, wcat2)


def _layer2(parts, ws2t, bs2, n, bn):
    d = parts.shape[2]

    def body(p_ref, w_ref, b_ref, o_ref):
        ssum = p_ref[0] + p_ref[1]
        o_ref[...] = ssum + jnp.dot(
            ssum, w_ref[...], preferred_element_type=jnp.float32) + b_ref[...]

    return pl.pallas_call(
        body,
        grid=(n // bn,),
        in_specs=[pl.BlockSpec((2, bn, d), lambda i: (0, i, 0)),
                  pl.BlockSpec((d, d), lambda i: (0, 0)),
                  pl.BlockSpec((1, d), lambda i: (0, 0))],
        out_specs=pl.BlockSpec((bn, d), lambda i: (i, 0)),
        out_shape=jax.ShapeDtypeStruct((n, d), jnp.float32),
    )(parts, ws2t, bs2)


def _mlp(g2, ws2t, bs2_row, wut, wmt, bp1, wp2t_pad, bp2_pad, user_row):
    gn, d = g2.shape
    half = gn // 2

    def body(g_ref, wi_ref, b2_ref, wu_ref, wm_ref, b1_ref, w2_ref, b3_ref,
             o_ref):
        ssum = g_ref[:half, :] + g_ref[half:, :]
        emb = ssum + jnp.dot(ssum, wi_ref[...],
                             preferred_element_type=jnp.float32) + b2_ref[...]
        u = jnp.dot(emb[user_row:user_row + 1, :], wu_ref[...],
                    preferred_element_type=jnp.float32)
        hidden = jnp.maximum(
            jnp.dot(emb, wm_ref[...], preferred_element_type=jnp.float32)
            + u + b1_ref[...], 0.0)
        o_ref[...] = jnp.dot(hidden, w2_ref[...],
                             preferred_element_type=jnp.float32) + b3_ref[...]

    return pl.pallas_call(
        body,
        out_shape=jax.ShapeDtypeStruct((half, d), jnp.float32),
    )(g2, ws2t, bs2_row, wut, wmt, bp1, wp2t_pad, bp2_pad)


def kernel(edge_index, edge_type, user_idx, movie_indices, node_emb,
           Wr1, Wr2, Ws1, bs1, Ws2, bs2, Wp1, bp1, Wp2, bp2):
    n, d = node_emb.shape
    e = edge_type.shape[0]
    r = Wr1.shape[0]
    m = movie_indices.shape[0]

    n_pad = 10240 if n == 10000 else ((n + NS * 64 - 1) // (NS * 64)) * NS * 64
    if n_pad <= n:
        n_pad = n + NS * 64
    bn = n // 10

    n_chunks = -(-e // (NW * CHUNK))
    n_chunks = ((n_chunks + 7) // 8) * 8
    e_pad = NW * CHUNK * n_chunks

    src = edge_index[0].astype(jnp.int32)
    dst = edge_index[1].astype(jnp.int32)
    et = edge_type.astype(jnp.int32)
    gidx = src * r + et
    pad_i = jnp.arange(e_pad - e, dtype=jnp.int32)
    gidx2d = jnp.concatenate(
        [gidx, pad_i % (n * r)]).reshape(NW * n_chunks, CHUNK)
    didx2d = jnp.concatenate(
        [dst, n + pad_i % (n_pad - n)]).reshape(NW * n_chunks, CHUNK)

    zeros_pad = jnp.zeros((n_pad, d), jnp.float32)

    wcat1 = jnp.concatenate([Wr1[i].T for i in range(r)], axis=1)
    wcat2 = jnp.concatenate([Wr2[i].T for i in range(r)], axis=1)

    table1 = _mm(node_emb, wcat1, bn).reshape(n * r, d)
    sl1 = _mm_bias(node_emb, Ws1.T, bs1.reshape(1, d), bn)
    parts1 = _edge_agg(table1, gidx2d, didx2d, zeros_pad, n_pad, d, n_chunks)

    table2 = _layer1b(parts1, sl1, wcat2, bn).reshape(n * r, d)
    parts2 = _edge_agg(table2, gidx2d, didx2d, zeros_pad, n_pad, d, n_chunks)

    hfinal = _layer2(parts2, Ws2.T, bs2.reshape(1, d), n, bn)

    g_rows = NW * (-(-(m + 1) // NW))
    g_rows = max(g_rows, NW)
    idx3 = jnp.concatenate([
        movie_indices.astype(jnp.int32),
        jnp.asarray(user_idx, jnp.int32).reshape(1),
        jnp.zeros((g_rows - m - 1,), jnp.int32),
    ])
    idx6 = jnp.concatenate([idx3, idx3 + n_pad])
    g2 = _gather_rows(parts2.reshape(NC * n_pad, d), idx6, d, 2 * g_rows // NW)

    wut = Wp1[:, :d].T
    wmt = Wp1[:, d:].T
    wp2t_pad = jnp.concatenate(
        [Wp2.T, jnp.zeros((d, d - 1), jnp.float32)], axis=1)
    bp2_pad = jnp.broadcast_to(bp2.reshape(1, 1), (1, d))
    scores_pad = _mlp(g2, Ws2.T, bs2.reshape(1, d), wut, wmt,
                      bp1.reshape(1, d), wp2t_pad, bp2_pad, m)

    return scores_pad[:m, 0], hfinal


# speedup vs baseline: 1.2406x; 1.2406x over previous
import functools

import jax
import jax.numpy as jnp
from jax import lax
from jax.experimental import pallas as pl
from jax.experimental.pallas import tpu as pltpu
from jax.experimental.pallas import tpu_sc as plsc

NC = 2
NS = 16
NW = NC * NS
CHUNK = 128


def _edge_agg(table, gidx2d, didx2d, zeros_pad, n_pad, d, n_chunks):
    rpt = n_pad // NS
    mesh = plsc.VectorSubcoreMesh(core_axis_name="c", subcore_axis_name="s")

    @functools.partial(
        pl.kernel,
        out_type=jax.ShapeDtypeStruct((NC, n_pad, d), jnp.float32),
        mesh=mesh,
        scratch_types=[
            pltpu.VMEM((n_chunks, CHUNK), jnp.int32),
            pltpu.VMEM((2, 8, CHUNK), jnp.int32),
            pltpu.VMEM((CHUNK, d), jnp.float32),
            pltpu.VMEM((CHUNK, d), jnp.float32),
            pltpu.VMEM_SHARED((n_pad, d), jnp.float32),
            pltpu.SemaphoreType.DMA,
            pltpu.SemaphoreType.DMA,
            pltpu.SemaphoreType.DMA,
        ],
    )
    def agg(table_hbm, gidx_hbm, didx_hbm, zeros_hbm, out_hbm,
            gidx_v, dring, rows_a, rows_b, acc,
            sem_ga, sem_gb, sem_i):
        c = lax.axis_index("c")
        s = lax.axis_index("s")
        wid = s * NC + c
        pltpu.sync_copy(zeros_hbm.at[pl.ds(s * rpt, rpt)],
                        acc.at[pl.ds(s * rpt, rpt)])
        pltpu.sync_copy(gidx_hbm.at[pl.ds(wid * n_chunks, n_chunks)], gidx_v)
        pltpu.sync_copy(didx_hbm.at[pl.ds(wid * n_chunks, 8)], dring.at[0])
        plsc.subcore_barrier()

        pltpu.async_copy(table_hbm.at[gidx_v.at[0]], rows_a, sem_ga)

        def step(t, carry):
            j = 2 * t
            grp = t >> 2
            q = grp & 1

            @pl.when((t & 3) == 0)
            def _():
                @pl.when(t > 0)
                def _():
                    pltpu.make_async_copy(
                        didx_hbm.at[pl.ds(0, 8)], dring.at[0], sem_i).wait()

                @pl.when(j + 8 < n_chunks)
                def _():
                    off = pl.multiple_of(wid * n_chunks + j + 8, 8)
                    pltpu.async_copy(didx_hbm.at[pl.ds(off, 8)],
                                     dring.at[1 - q], sem_i)

            pltpu.async_copy(table_hbm.at[gidx_v.at[j + 1]], rows_b, sem_gb)
            pltpu.make_async_copy(table_hbm.at[gidx_v.at[0]], rows_a,
                                  sem_ga).wait()
            pltpu.sync_copy(rows_a, acc.at[dring.at[q, 2 * (t & 3)]],
                            add=True)

            @pl.when(j + 2 < n_chunks)
            def _():
                pltpu.async_copy(table_hbm.at[gidx_v.at[j + 2]], rows_a,
                                 sem_ga)

            pltpu.make_async_copy(table_hbm.at[gidx_v.at[0]], rows_b,
                                  sem_gb).wait()
            pltpu.sync_copy(rows_b, acc.at[dring.at[q, 2 * (t & 3) + 1]],
                            add=True)
            return carry

        lax.fori_loop(0, n_chunks // 2, step, 0)
        plsc.subcore_barrier()
        pltpu.sync_copy(acc.at[pl.ds(s * rpt, rpt)],
                        out_hbm.at[c, pl.ds(s * rpt, rpt)])

    return agg(table, gidx2d, didx2d, zeros_pad)


def _gather_rows(table, idx, d, per_w):
    g = idx.shape[0]
    mesh = plsc.VectorSubcoreMesh(core_axis_name="c", subcore_axis_name="s")

    @functools.partial(
        pl.kernel,
        out_type=jax.ShapeDtypeStruct((g, d), jnp.float32),
        mesh=mesh,
        scratch_types=[
            pltpu.VMEM((per_w,), jnp.int32),
            pltpu.VMEM((per_w, d), jnp.float32),
            pltpu.SemaphoreType.DMA,
        ],
    )
    def gat(table_hbm, idx_hbm, out_hbm, idx_v, rows_v, sem):
        wid = lax.axis_index("s") * NC + lax.axis_index("c")
        base = wid * per_w
        pltpu.sync_copy(idx_hbm.at[pl.ds(base, per_w)], idx_v)
        pltpu.async_copy(table_hbm.at[idx_v], rows_v, sem).wait()
        pltpu.sync_copy(rows_v, out_hbm.at[pl.ds(base, per_w)])

    return gat(table, idx)


def _mm_rm(x, wstack, bn):
    n, d = x.shape
    r = wstack.shape[0]

    def body(x_ref, w_ref, o_ref):
        o_ref[...] = jnp.dot(x_ref[...], w_ref[0],
                             preferred_element_type=jnp.float32)[None]

    return pl.pallas_call(
        body,
        grid=(n // bn, r),
        in_specs=[pl.BlockSpec((bn, d), lambda ni, ri: (ni, 0)),
                  pl.BlockSpec((1, d, d), lambda ni, ri: (ri, 0, 0))],
        out_specs=pl.BlockSpec((1, bn, d), lambda ni, ri: (ri, ni, 0)),
        out_shape=jax.ShapeDtypeStruct((r, n, d), jnp.float32),
    )(x, wstack)


def _mm_bias(x, w, b, bn):
    n, k = x.shape
    m = w.shape[1]

    def body(x_ref, w_ref, b_ref, o_ref):
        o_ref[...] = jnp.dot(x_ref[...], w_ref[...],
                             preferred_element_type=jnp.float32) + b_ref[...]

    return pl.pallas_call(
        body,
        grid=(n // bn,),
        in_specs=[pl.BlockSpec((bn, k), lambda i: (i, 0)),
                  pl.BlockSpec((k, m), lambda i: (0, 0)),
                  pl.BlockSpec((1, m), lambda i: (0, 0))],
        out_specs=pl.BlockSpec((bn, m), lambda i: (i, 0)),
        out_shape=jax.ShapeDtypeStruct((n, m), jnp.float32),
    )(x, w, b)


def _layer1b_rm(parts, sl, wstack2, bn):
    n, d = sl.shape
    r = wstack2.shape[0]

    def body(p_ref, sl_ref, w_ref, o_ref):
        h = jnp.maximum(p_ref[0] + p_ref[1] + sl_ref[...], 0.0)
        o_ref[...] = jnp.dot(h, w_ref[0],
                             preferred_element_type=jnp.float32)[None]

    return pl.pallas_call(
        body,
        grid=(n // bn, r),
        in_specs=[pl.BlockSpec((2, bn, d), lambda ni, ri: (0, ni, 0)),
                  pl.BlockSpec((bn, d), lambda ni, ri: (ni, 0)),
                  pl.BlockSpec((1, d, d), lambda ni, ri: (ri, 0, 0))],
        out_specs=pl.BlockSpec((1, bn, d), lambda ni, ri: (ri, ni, 0)),
        out_shape=jax.ShapeDtypeStruct((r, n, d), jnp.float32),
    )(parts, sl, wstack2)


def _layer2(parts, ws2t, bs2, n, bn):
    d = parts.shape[2]

    def body(p_ref, w_ref, b_ref, o_ref):
        ssum = p_ref[0] + p_ref[1]
        o_ref[...] = ssum + jnp.dot(
            ssum, w_ref[...], preferred_element_type=jnp.float32) + b_ref[...]

    return pl.pallas_call(
        body,
        grid=(n // bn,),
        in_specs=[pl.BlockSpec((2, bn, d), lambda i: (0, i, 0)),
                  pl.BlockSpec((d, d), lambda i: (0, 0)),
                  pl.BlockSpec((1, d), lambda i: (0, 0))],
        out_specs=pl.BlockSpec((bn, d), lambda i: (i, 0)),
        out_shape=jax.ShapeDtypeStruct((n, d), jnp.float32),
    )(parts, ws2t, bs2)


def _mlp(g2, ws2t, bs2_row, wut, wmt, bp1, wp2t_pad, bp2_pad, user_row):
    gn, d = g2.shape
    half = gn // 2

    def body(g_ref, wi_ref, b2_ref, wu_ref, wm_ref, b1_ref, w2_ref, b3_ref,
             o_ref):
        ssum = g_ref[:half, :] + g_ref[half:, :]
        emb = ssum + jnp.dot(ssum, wi_ref[...],
                             preferred_element_type=jnp.float32) + b2_ref[...]
        u = jnp.dot(emb[user_row:user_row + 1, :], wu_ref[...],
                    preferred_element_type=jnp.float32)
        hidden = jnp.maximum(
            jnp.dot(emb, wm_ref[...], preferred_element_type=jnp.float32)
            + u + b1_ref[...], 0.0)
        o_ref[...] = jnp.dot(hidden, w2_ref[...],
                             preferred_element_type=jnp.float32) + b3_ref[...]

    return pl.pallas_call(
        body,
        out_shape=jax.ShapeDtypeStruct((half, d), jnp.float32),
    )(g2, ws2t, bs2_row, wut, wmt, bp1, wp2t_pad, bp2_pad)


def kernel(edge_index, edge_type, user_idx, movie_indices, node_emb,
           Wr1, Wr2, Ws1, bs1, Ws2, bs2, Wp1, bp1, Wp2, bp2):
    n, d = node_emb.shape
    e = edge_type.shape[0]
    r = Wr1.shape[0]
    m = movie_indices.shape[0]

    n_pad = 10240 if n == 10000 else ((n + NS * 64 - 1) // (NS * 64)) * NS * 64
    if n_pad <= n:
        n_pad = n + NS * 64
    bn = n // 10

    n_chunks = -(-e // (NW * CHUNK))
    n_chunks = ((n_chunks + 7) // 8) * 8
    e_pad = NW * CHUNK * n_chunks

    src = edge_index[0].astype(jnp.int32)
    dst = edge_index[1].astype(jnp.int32)
    et = edge_type.astype(jnp.int32)
    gidx = et * n + src
    pad_i = jnp.arange(e_pad - e, dtype=jnp.int32)
    gidx2d = jnp.concatenate(
        [gidx, pad_i % (n * r)]).reshape(NW * n_chunks, CHUNK)
    didx2d = jnp.concatenate(
        [dst, n + pad_i % (n_pad - n)]).reshape(NW * n_chunks, CHUNK)

    zeros_pad = jnp.zeros((n_pad, d), jnp.float32)

    wstack1 = Wr1.transpose(0, 2, 1)
    wstack2 = Wr2.transpose(0, 2, 1)

    table1 = _mm_rm(node_emb, wstack1, bn).reshape(r * n, d)
    sl1 = _mm_bias(node_emb, Ws1.T, bs1.reshape(1, d), bn)
    parts1 = _edge_agg(table1, gidx2d, didx2d, zeros_pad, n_pad, d, n_chunks)

    table2 = _layer1b_rm(parts1, sl1, wstack2, bn).reshape(r * n, d)
    parts2 = _edge_agg(table2, gidx2d, didx2d, zeros_pad, n_pad, d, n_chunks)

    hfinal = _layer2(parts2, Ws2.T, bs2.reshape(1, d), n, bn)

    g_rows = NW * (-(-(m + 1) // NW))
    g_rows = max(g_rows, NW)
    idx3 = jnp.concatenate([
        movie_indices.astype(jnp.int32),
        jnp.asarray(user_idx, jnp.int32).reshape(1),
        jnp.zeros((g_rows - m - 1,), jnp.int32),
    ])
    idx6 = jnp.concatenate([idx3, idx3 + n_pad])
    g2 = _gather_rows(parts2.reshape(NC * n_pad, d), idx6, d, 2 * g_rows // NW)

    wut = Wp1[:, :d].T
    wmt = Wp1[:, d:].T
    wp2t_pad = jnp.concatenate(
        [Wp2.T, jnp.zeros((d, d - 1), jnp.float32)], axis=1)
    bp2_pad = jnp.broadcast_to(bp2.reshape(1, 1), (1, d))
    scores_pad = _mlp(g2, Ws2.T, bs2.reshape(1, d), wut, wmt,
                      bp1.reshape(1, d), wp2t_pad, bp2_pad, m)

    return scores_pad[:m, 0], hfinal


# speedup vs baseline: 1.3763x; 1.1094x over previous
import functools

import jax
import jax.numpy as jnp
from jax import lax
from jax.experimental import pallas as pl
from jax.experimental.pallas import tpu as pltpu
from jax.experimental.pallas import tpu_sc as plsc

NC = 2
NS = 16
NW = NC * NS
CHUNK = 128


def _edge_agg(table, gidx2d, didx2d, zeros_pad, n_pad, d, n_chunks):
    rpt = n_pad // NS
    mesh = plsc.VectorSubcoreMesh(core_axis_name="c", subcore_axis_name="s")

    @functools.partial(
        pl.kernel,
        out_type=jax.ShapeDtypeStruct((NC, n_pad, d), jnp.float32),
        mesh=mesh,
        scratch_types=[
            pltpu.VMEM((n_chunks, CHUNK), jnp.int32),
            pltpu.VMEM((2, 8, CHUNK), jnp.int32),
            pltpu.VMEM((CHUNK, d), jnp.float32),
            pltpu.VMEM((CHUNK, d), jnp.float32),
            pltpu.VMEM_SHARED((n_pad, d), jnp.float32),
            pltpu.SemaphoreType.DMA,
            pltpu.SemaphoreType.DMA,
            pltpu.SemaphoreType.DMA,
        ],
    )
    def agg(table_hbm, gidx_hbm, didx_hbm, zeros_hbm, out_hbm,
            gidx_v, dring, rows_a, rows_b, acc,
            sem_ga, sem_gb, sem_i):
        c = lax.axis_index("c")
        s = lax.axis_index("s")
        wid = s * NC + c
        pltpu.sync_copy(zeros_hbm.at[pl.ds(s * rpt, rpt)],
                        acc.at[pl.ds(s * rpt, rpt)])
        pltpu.sync_copy(gidx_hbm.at[pl.ds(wid * n_chunks, n_chunks)], gidx_v)
        pltpu.sync_copy(didx_hbm.at[pl.ds(wid * n_chunks, 8)], dring.at[0])
        plsc.subcore_barrier()

        pltpu.async_copy(table_hbm.at[gidx_v.at[0]], rows_a, sem_ga)

        def step(t, carry):
            j = 2 * t
            grp = t >> 2
            q = grp & 1

            @pl.when((t & 3) == 0)
            def _():
                @pl.when(t > 0)
                def _():
                    pltpu.make_async_copy(
                        didx_hbm.at[pl.ds(0, 8)], dring.at[0], sem_i).wait()

                @pl.when(j + 8 < n_chunks)
                def _():
                    off = pl.multiple_of(wid * n_chunks + j + 8, 8)
                    pltpu.async_copy(didx_hbm.at[pl.ds(off, 8)],
                                     dring.at[1 - q], sem_i)

            pltpu.async_copy(table_hbm.at[gidx_v.at[j + 1]], rows_b, sem_gb)
            pltpu.make_async_copy(table_hbm.at[gidx_v.at[0]], rows_a,
                                  sem_ga).wait()
            pltpu.sync_copy(rows_a, acc.at[dring.at[q, 2 * (t & 3)]],
                            add=True)

            @pl.when(j + 2 < n_chunks)
            def _():
                pltpu.async_copy(table_hbm.at[gidx_v.at[j + 2]], rows_a,
                                 sem_ga)

            pltpu.make_async_copy(table_hbm.at[gidx_v.at[0]], rows_b,
                                  sem_gb).wait()
            pltpu.sync_copy(rows_b, acc.at[dring.at[q, 2 * (t & 3) + 1]],
                            add=True)
            return carry

        lax.fori_loop(0, n_chunks // 2, step, 0)
        plsc.subcore_barrier()
        pltpu.sync_copy(acc.at[pl.ds(s * rpt, rpt)],
                        out_hbm.at[c, pl.ds(s * rpt, rpt)])

    return agg(table, gidx2d, didx2d, zeros_pad)


def _gather_rows(table, idx, d, per_w):
    g = idx.shape[0]
    mesh = plsc.VectorSubcoreMesh(core_axis_name="c", subcore_axis_name="s")

    @functools.partial(
        pl.kernel,
        out_type=jax.ShapeDtypeStruct((g, d), jnp.float32),
        mesh=mesh,
        scratch_types=[
            pltpu.VMEM((per_w,), jnp.int32),
            pltpu.VMEM((per_w, d), jnp.float32),
            pltpu.SemaphoreType.DMA,
        ],
    )
    def gat(table_hbm, idx_hbm, out_hbm, idx_v, rows_v, sem):
        wid = lax.axis_index("s") * NC + lax.axis_index("c")
        base = wid * per_w
        pltpu.sync_copy(idx_hbm.at[pl.ds(base, per_w)], idx_v)
        pltpu.async_copy(table_hbm.at[idx_v], rows_v, sem).wait()
        pltpu.sync_copy(rows_v, out_hbm.at[pl.ds(base, per_w)])

    return gat(table, idx)


def _mm_rm(x, wstack, bn):
    n, d = x.shape
    r = wstack.shape[0]

    def body(x_ref, w_ref, o_ref):
        xx = x_ref[...]
        for ri in range(r):
            o_ref[ri] = jnp.dot(xx, w_ref[ri],
                                preferred_element_type=jnp.float32)

    return pl.pallas_call(
        body,
        grid=(n // bn,),
        in_specs=[pl.BlockSpec((bn, d), lambda ni: (ni, 0)),
                  pl.BlockSpec((r, d, d), lambda ni: (0, 0, 0))],
        out_specs=pl.BlockSpec((r, bn, d), lambda ni: (0, ni, 0)),
        out_shape=jax.ShapeDtypeStruct((r, n, d), jnp.float32),
    )(x, wstack)


def _mm_bias(x, w, b, bn):
    n, k = x.shape
    m = w.shape[1]

    def body(x_ref, w_ref, b_ref, o_ref):
        o_ref[...] = jnp.dot(x_ref[...], w_ref[...],
                             preferred_element_type=jnp.float32) + b_ref[...]

    return pl.pallas_call(
        body,
        grid=(n // bn,),
        in_specs=[pl.BlockSpec((bn, k), lambda i: (i, 0)),
                  pl.BlockSpec((k, m), lambda i: (0, 0)),
                  pl.BlockSpec((1, m), lambda i: (0, 0))],
        out_specs=pl.BlockSpec((bn, m), lambda i: (i, 0)),
        out_shape=jax.ShapeDtypeStruct((n, m), jnp.float32),
    )(x, w, b)


def _layer1b_rm(parts, sl, wstack2, bn):
    n, d = sl.shape
    r = wstack2.shape[0]

    def body(p_ref, sl_ref, w_ref, o_ref):
        h = jnp.maximum(p_ref[0] + p_ref[1] + sl_ref[...], 0.0)
        for ri in range(r):
            o_ref[ri] = jnp.dot(h, w_ref[ri],
                                preferred_element_type=jnp.float32)

    return pl.pallas_call(
        body,
        grid=(n // bn,),
        in_specs=[pl.BlockSpec((2, bn, d), lambda ni: (0, ni, 0)),
                  pl.BlockSpec((bn, d), lambda ni: (ni, 0)),
                  pl.BlockSpec((r, d, d), lambda ni: (0, 0, 0))],
        out_specs=pl.BlockSpec((r, bn, d), lambda ni: (0, ni, 0)),
        out_shape=jax.ShapeDtypeStruct((r, n, d), jnp.float32),
    )(parts, sl, wstack2)


def _layer2(parts, ws2t, bs2, n, bn):
    d = parts.shape[2]

    def body(p_ref, w_ref, b_ref, o_ref):
        ssum = p_ref[0] + p_ref[1]
        o_ref[...] = ssum + jnp.dot(
            ssum, w_ref[...], preferred_element_type=jnp.float32) + b_ref[...]

    return pl.pallas_call(
        body,
        grid=(n // bn,),
        in_specs=[pl.BlockSpec((2, bn, d), lambda i: (0, i, 0)),
                  pl.BlockSpec((d, d), lambda i: (0, 0)),
                  pl.BlockSpec((1, d), lambda i: (0, 0))],
        out_specs=pl.BlockSpec((bn, d), lambda i: (i, 0)),
        out_shape=jax.ShapeDtypeStruct((n, d), jnp.float32),
    )(parts, ws2t, bs2)


def _mlp(g2, ws2t, bs2_row, wut, wmt, bp1, wp2t_pad, bp2_pad, user_row):
    gn, d = g2.shape
    half = gn // 2

    def body(g_ref, wi_ref, b2_ref, wu_ref, wm_ref, b1_ref, w2_ref, b3_ref,
             o_ref):
        ssum = g_ref[:half, :] + g_ref[half:, :]
        emb = ssum + jnp.dot(ssum, wi_ref[...],
                             preferred_element_type=jnp.float32) + b2_ref[...]
        u = jnp.dot(emb[user_row:user_row + 1, :], wu_ref[...],
                    preferred_element_type=jnp.float32)
        hidden = jnp.maximum(
            jnp.dot(emb, wm_ref[...], preferred_element_type=jnp.float32)
            + u + b1_ref[...], 0.0)
        o_ref[...] = jnp.dot(hidden, w2_ref[...],
                             preferred_element_type=jnp.float32) + b3_ref[...]

    return pl.pallas_call(
        body,
        out_shape=jax.ShapeDtypeStruct((half, d), jnp.float32),
    )(g2, ws2t, bs2_row, wut, wmt, bp1, wp2t_pad, bp2_pad)


def kernel(edge_index, edge_type, user_idx, movie_indices, node_emb,
           Wr1, Wr2, Ws1, bs1, Ws2, bs2, Wp1, bp1, Wp2, bp2):
    n, d = node_emb.shape
    e = edge_type.shape[0]
    r = Wr1.shape[0]
    m = movie_indices.shape[0]

    n_pad = 10240 if n == 10000 else ((n + NS * 64 - 1) // (NS * 64)) * NS * 64
    if n_pad <= n:
        n_pad = n + NS * 64
    bn = n // 10

    n_chunks = -(-e // (NW * CHUNK))
    n_chunks = ((n_chunks + 7) // 8) * 8
    e_pad = NW * CHUNK * n_chunks

    src = edge_index[0].astype(jnp.int32)
    dst = edge_index[1].astype(jnp.int32)
    et = edge_type.astype(jnp.int32)
    gidx = et * n + src
    pad_i = jnp.arange(e_pad - e, dtype=jnp.int32)
    gidx2d = jnp.concatenate(
        [gidx, pad_i % (n * r)]).reshape(NW * n_chunks, CHUNK)
    didx2d = jnp.concatenate(
        [dst, n + pad_i % (n_pad - n)]).reshape(NW * n_chunks, CHUNK)

    zeros_pad = jnp.zeros((n_pad, d), jnp.float32)

    wstack1 = Wr1.transpose(0, 2, 1)
    wstack2 = Wr2.transpose(0, 2, 1)

    table1 = _mm_rm(node_emb, wstack1, bn).reshape(r * n, d)
    sl1 = _mm_bias(node_emb, Ws1.T, bs1.reshape(1, d), bn)
    parts1 = _edge_agg(table1, gidx2d, didx2d, zeros_pad, n_pad, d, n_chunks)

    table2 = _layer1b_rm(parts1, sl1, wstack2, bn).reshape(r * n, d)
    parts2 = _edge_agg(table2, gidx2d, didx2d, zeros_pad, n_pad, d, n_chunks)

    hfinal = _layer2(parts2, Ws2.T, bs2.reshape(1, d), n, bn)

    g_rows = NW * (-(-(m + 1) // NW))
    g_rows = max(g_rows, NW)
    idx3 = jnp.concatenate([
        movie_indices.astype(jnp.int32),
        jnp.asarray(user_idx, jnp.int32).reshape(1),
        jnp.zeros((g_rows - m - 1,), jnp.int32),
    ])
    idx6 = jnp.concatenate([idx3, idx3 + n_pad])
    g2 = _gather_rows(parts2.reshape(NC * n_pad, d), idx6, d, 2 * g_rows // NW)

    wut = Wp1[:, :d].T
    wmt = Wp1[:, d:].T
    wp2t_pad = jnp.concatenate(
        [Wp2.T, jnp.zeros((d, d - 1), jnp.float32)], axis=1)
    bp2_pad = jnp.broadcast_to(bp2.reshape(1, 1), (1, d))
    scores_pad = _mlp(g2, Ws2.T, bs2.reshape(1, d), wut, wmt,
                      bp1.reshape(1, d), wp2t_pad, bp2_pad, m)

    return scores_pad[:m, 0], hfinal


# speedup vs baseline: 1.4382x; 1.0449x over previous
import functools

import jax
import jax.numpy as jnp
from jax import lax
from jax.experimental import pallas as pl
from jax.experimental.pallas import tpu as pltpu
from jax.experimental.pallas import tpu_sc as plsc

NC = 2
NS = 16
NW = NC * NS
CHUNK = 128


def _edge_agg(table, gidx2d, didx2d, zeros_pad, n_pad, d, n_chunks):
    rpt = n_pad // NS
    mesh = plsc.VectorSubcoreMesh(core_axis_name="c", subcore_axis_name="s")

    @functools.partial(
        pl.kernel,
        out_type=jax.ShapeDtypeStruct((NC, n_pad, d), jnp.float32),
        mesh=mesh,
        scratch_types=[
            pltpu.VMEM((n_chunks, CHUNK), jnp.int32),
            pltpu.VMEM((2, 8, CHUNK), jnp.int32),
            pltpu.VMEM((CHUNK, d), jnp.float32),
            pltpu.VMEM((CHUNK, d), jnp.float32),
            pltpu.VMEM_SHARED((n_pad, d), jnp.float32),
            pltpu.SemaphoreType.DMA,
            pltpu.SemaphoreType.DMA,
            pltpu.SemaphoreType.DMA,
        ],
    )
    def agg(table_hbm, gidx_hbm, didx_hbm, zeros_hbm, out_hbm,
            gidx_v, dring, rows_a, rows_b, acc,
            sem_ga, sem_gb, sem_i):
        c = lax.axis_index("c")
        s = lax.axis_index("s")
        wid = s * NC + c
        pltpu.sync_copy(gidx_hbm.at[pl.ds(wid * n_chunks, n_chunks)], gidx_v)
        pltpu.async_copy(table_hbm.at[gidx_v.at[0]], rows_a, sem_ga)
        pltpu.sync_copy(didx_hbm.at[pl.ds(wid * n_chunks, 8)], dring.at[0])
        pltpu.sync_copy(zeros_hbm.at[pl.ds(s * rpt, rpt)],
                        acc.at[pl.ds(s * rpt, rpt)])
        plsc.subcore_barrier()


        def step(t, carry):
            j = 2 * t
            grp = t >> 2
            q = grp & 1

            @pl.when((t & 3) == 0)
            def _():
                @pl.when(t > 0)
                def _():
                    pltpu.make_async_copy(
                        didx_hbm.at[pl.ds(0, 8)], dring.at[0], sem_i).wait()

                @pl.when(j + 8 < n_chunks)
                def _():
                    off = pl.multiple_of(wid * n_chunks + j + 8, 8)
                    pltpu.async_copy(didx_hbm.at[pl.ds(off, 8)],
                                     dring.at[1 - q], sem_i)

            pltpu.async_copy(table_hbm.at[gidx_v.at[j + 1]], rows_b, sem_gb)
            pltpu.make_async_copy(table_hbm.at[gidx_v.at[0]], rows_a,
                                  sem_ga).wait()
            pltpu.sync_copy(rows_a, acc.at[dring.at[q, 2 * (t & 3)]],
                            add=True)

            @pl.when(j + 2 < n_chunks)
            def _():
                pltpu.async_copy(table_hbm.at[gidx_v.at[j + 2]], rows_a,
                                 sem_ga)

            pltpu.make_async_copy(table_hbm.at[gidx_v.at[0]], rows_b,
                                  sem_gb).wait()
            pltpu.sync_copy(rows_b, acc.at[dring.at[q, 2 * (t & 3) + 1]],
                            add=True)
            return carry

        lax.fori_loop(0, n_chunks // 2, step, 0)
        plsc.subcore_barrier()
        pltpu.sync_copy(acc.at[pl.ds(s * rpt, rpt)],
                        out_hbm.at[c, pl.ds(s * rpt, rpt)])

    return agg(table, gidx2d, didx2d, zeros_pad)


def _gather_rows(table, idx, d, per_w):
    g = idx.shape[0]
    mesh = plsc.VectorSubcoreMesh(core_axis_name="c", subcore_axis_name="s")

    @functools.partial(
        pl.kernel,
        out_type=jax.ShapeDtypeStruct((g, d), jnp.float32),
        mesh=mesh,
        scratch_types=[
            pltpu.VMEM((per_w,), jnp.int32),
            pltpu.VMEM((per_w, d), jnp.float32),
            pltpu.SemaphoreType.DMA,
        ],
    )
    def gat(table_hbm, idx_hbm, out_hbm, idx_v, rows_v, sem):
        wid = lax.axis_index("s") * NC + lax.axis_index("c")
        base = wid * per_w
        pltpu.sync_copy(idx_hbm.at[pl.ds(base, per_w)], idx_v)
        pltpu.async_copy(table_hbm.at[idx_v], rows_v, sem).wait()
        pltpu.sync_copy(rows_v, out_hbm.at[pl.ds(base, per_w)])

    return gat(table, idx)


def _mm_rm(x, wstack, bn):
    n, d = x.shape
    r = wstack.shape[0]

    def body(x_ref, w_ref, o_ref):
        xx = x_ref[...]
        for ri in range(r):
            o_ref[ri] = jnp.dot(xx, w_ref[ri],
                                preferred_element_type=jnp.float32)

    return pl.pallas_call(
        body,
        grid=(n // bn,),
        in_specs=[pl.BlockSpec((bn, d), lambda ni: (ni, 0)),
                  pl.BlockSpec((r, d, d), lambda ni: (0, 0, 0))],
        out_specs=pl.BlockSpec((r, bn, d), lambda ni: (0, ni, 0)),
        out_shape=jax.ShapeDtypeStruct((r, n, d), jnp.float32),
    )(x, wstack)


def _mm_bias(x, w, b, bn):
    n, k = x.shape
    m = w.shape[1]

    def body(x_ref, w_ref, b_ref, o_ref):
        o_ref[...] = jnp.dot(x_ref[...], w_ref[...],
                             preferred_element_type=jnp.float32) + b_ref[...]

    return pl.pallas_call(
        body,
        grid=(n // bn,),
        in_specs=[pl.BlockSpec((bn, k), lambda i: (i, 0)),
                  pl.BlockSpec((k, m), lambda i: (0, 0)),
                  pl.BlockSpec((1, m), lambda i: (0, 0))],
        out_specs=pl.BlockSpec((bn, m), lambda i: (i, 0)),
        out_shape=jax.ShapeDtypeStruct((n, m), jnp.float32),
    )(x, w, b)


def _layer1b_rm(parts, sl, wstack2, bn):
    n, d = sl.shape
    r = wstack2.shape[0]

    def body(p_ref, sl_ref, w_ref, o_ref):
        h = jnp.maximum(p_ref[0] + p_ref[1] + sl_ref[...], 0.0)
        for ri in range(r):
            o_ref[ri] = jnp.dot(h, w_ref[ri],
                                preferred_element_type=jnp.float32)

    return pl.pallas_call(
        body,
        grid=(n // bn,),
        in_specs=[pl.BlockSpec((2, bn, d), lambda ni: (0, ni, 0)),
                  pl.BlockSpec((bn, d), lambda ni: (ni, 0)),
                  pl.BlockSpec((r, d, d), lambda ni: (0, 0, 0))],
        out_specs=pl.BlockSpec((r, bn, d), lambda ni: (0, ni, 0)),
        out_shape=jax.ShapeDtypeStruct((r, n, d), jnp.float32),
    )(parts, sl, wstack2)


def _layer2(parts, ws2t, bs2, n, bn):
    d = parts.shape[2]

    def body(p_ref, w_ref, b_ref, o_ref):
        ssum = p_ref[0] + p_ref[1]
        o_ref[...] = ssum + jnp.dot(
            ssum, w_ref[...], preferred_element_type=jnp.float32) + b_ref[...]

    return pl.pallas_call(
        body,
        grid=(n // bn,),
        in_specs=[pl.BlockSpec((2, bn, d), lambda i: (0, i, 0)),
                  pl.BlockSpec((d, d), lambda i: (0, 0)),
                  pl.BlockSpec((1, d), lambda i: (0, 0))],
        out_specs=pl.BlockSpec((bn, d), lambda i: (i, 0)),
        out_shape=jax.ShapeDtypeStruct((n, d), jnp.float32),
    )(parts, ws2t, bs2)


def _mlp(g2, ws2t, bs2_row, wut, wmt, bp1, wp2t_pad, bp2_pad, user_row):
    gn, d = g2.shape
    half = gn // 2

    def body(g_ref, wi_ref, b2_ref, wu_ref, wm_ref, b1_ref, w2_ref, b3_ref,
             o_ref):
        ssum = g_ref[:half, :] + g_ref[half:, :]
        emb = ssum + jnp.dot(ssum, wi_ref[...],
                             preferred_element_type=jnp.float32) + b2_ref[...]
        u = jnp.dot(emb[user_row:user_row + 1, :], wu_ref[...],
                    preferred_element_type=jnp.float32)
        hidden = jnp.maximum(
            jnp.dot(emb, wm_ref[...], preferred_element_type=jnp.float32)
            + u + b1_ref[...], 0.0)
        o_ref[...] = jnp.dot(hidden, w2_ref[...],
                             preferred_element_type=jnp.float32) + b3_ref[...]

    return pl.pallas_call(
        body,
        out_shape=jax.ShapeDtypeStruct((half, d), jnp.float32),
    )(g2, ws2t, bs2_row, wut, wmt, bp1, wp2t_pad, bp2_pad)


def kernel(edge_index, edge_type, user_idx, movie_indices, node_emb,
           Wr1, Wr2, Ws1, bs1, Ws2, bs2, Wp1, bp1, Wp2, bp2):
    n, d = node_emb.shape
    e = edge_type.shape[0]
    r = Wr1.shape[0]
    m = movie_indices.shape[0]

    n_pad = 10240 if n == 10000 else ((n + NS * 64 - 1) // (NS * 64)) * NS * 64
    if n_pad <= n:
        n_pad = n + NS * 64
    bn = n // 5

    n_chunks = -(-e // (NW * CHUNK))
    n_chunks = ((n_chunks + 7) // 8) * 8
    e_pad = NW * CHUNK * n_chunks

    src = edge_index[0].astype(jnp.int32)
    dst = edge_index[1].astype(jnp.int32)
    et = edge_type.astype(jnp.int32)
    gidx = et * n + src
    pad_i = jnp.arange(e_pad - e, dtype=jnp.int32)
    gidx2d = jnp.concatenate(
        [gidx, pad_i % (n * r)]).reshape(NW * n_chunks, CHUNK)
    didx2d = jnp.concatenate(
        [dst, n + pad_i % (n_pad - n)]).reshape(NW * n_chunks, CHUNK)

    zeros_pad = jnp.zeros((n_pad, d), jnp.float32)

    wstack1 = Wr1.transpose(0, 2, 1)
    wstack2 = Wr2.transpose(0, 2, 1)

    table1 = _mm_rm(node_emb, wstack1, bn).reshape(r * n, d)
    sl1 = _mm_bias(node_emb, Ws1.T, bs1.reshape(1, d), bn)
    parts1 = _edge_agg(table1, gidx2d, didx2d, zeros_pad, n_pad, d, n_chunks)

    table2 = _layer1b_rm(parts1, sl1, wstack2, bn).reshape(r * n, d)
    parts2 = _edge_agg(table2, gidx2d, didx2d, zeros_pad, n_pad, d, n_chunks)

    hfinal = _layer2(parts2, Ws2.T, bs2.reshape(1, d), n, bn)

    g_rows = NW * (-(-(m + 1) // NW))
    g_rows = max(g_rows, NW)
    idx3 = jnp.concatenate([
        movie_indices.astype(jnp.int32),
        jnp.asarray(user_idx, jnp.int32).reshape(1),
        jnp.zeros((g_rows - m - 1,), jnp.int32),
    ])
    idx6 = jnp.concatenate([idx3, idx3 + n_pad])
    g2 = _gather_rows(parts2.reshape(NC * n_pad, d), idx6, d, 2 * g_rows // NW)

    wut = Wp1[:, :d].T
    wmt = Wp1[:, d:].T
    wp2t_pad = jnp.concatenate(
        [Wp2.T, jnp.zeros((d, d - 1), jnp.float32)], axis=1)
    bp2_pad = jnp.broadcast_to(bp2.reshape(1, 1), (1, d))
    scores_pad = _mlp(g2, Ws2.T, bs2.reshape(1, d), wut, wmt,
                      bp1.reshape(1, d), wp2t_pad, bp2_pad, m)

    return scores_pad[:m, 0], hfinal
